# baseline (speedup 1.0000x reference)
import functools

import jax
import jax.numpy as jnp
from jax import lax
from jax.experimental import pallas as pl
from jax.experimental.pallas import tpu as pltpu
from jax.experimental.pallas import tpu_sc as plsc

N = 10000
E = 320000
D = 128
L = 5
NCLS = 120

NTILES = 32
CH = 128
NCHUNK = 80
EPT = NCHUNK * CH
EPAD = NTILES * EPT
NB = 10
BN = N // NB
AGG_ROWS = 10112
ZR = AGG_ROWS // 16
NP_EMB = NTILES * 3 * CH


def _prep_body(a_ref, b_ref, src_ref, xm0_ref, xm1_ref, gidx_ref, pidx_ref):
    gidx_ref[...] = (a_ref[...] * 4 + b_ref[...]) * N + src_ref[...]
    pidx_ref[...] = xm0_ref[...] * NCLS + xm1_ref[...]


def _pair_body(e0_ref, e1_ref, out_ref):
    t = e0_ref[...][:, None, :] + e1_ref[...][None, :, :]
    out_ref[...] = t.reshape(NCLS * NCLS, D)


def _hcat_body(h_ref, e0_ref, e1_ref, out_ref):
    c = pl.program_id(1)
    t = e0_ref[pl.ds(c // 4, 1), :] + e1_ref[pl.ds(c % 4, 1), :]
    out_ref[...] = jnp.maximum(h_ref[...] + t, 0.0)


def _mlp_body(pa_ref, pb_ref, h_ref, w1_ref, b1_ref, w2_ref, b2_ref, out_ref,
              *, relu_out):
    u = pa_ref[...] + pb_ref[...] + h_ref[...]
    z = jnp.maximum(
        jnp.dot(u, w1_ref[...], preferred_element_type=jnp.float32)
        + b1_ref[...], 0.0)
    o = jnp.dot(z, w2_ref[...], preferred_element_type=jnp.float32) + b2_ref[...]
    if relu_out:
        o = jnp.maximum(o, 0.0)
    out_ref[...] = o


def _mlp_mask_body(pa_ref, pb_ref, h_ref, w1_ref, b1_ref, w2_ref, b2_ref,
                   mk_ref, me_ref, out_ref):
    u = pa_ref[...] + pb_ref[...] + h_ref[...]
    z = jnp.maximum(
        jnp.dot(u, w1_ref[...], preferred_element_type=jnp.float32)
        + b1_ref[...], 0.0)
    o = jnp.dot(z, w2_ref[...], preferred_element_type=jnp.float32) + b2_ref[...]
    out_ref[...] = jnp.where(mk_ref[...] > 0, me_ref[...], o)


def _loss_body(hd_ref, wo_ref, bo_ref, x0_ref, mk_ref, out_ref, acc_ref):
    i = pl.program_id(0)
    logits = jnp.dot(hd_ref[...], wo_ref[...],
                     preferred_element_type=jnp.float32) + bo_ref[...]
    iota = lax.broadcasted_iota(jnp.int32, (BN, NCLS), 1)
    sel = jnp.sum(jnp.where(iota == x0_ref[...], logits, 0.0), axis=1)
    nrm = jnp.sqrt(jnp.sum(logits * logits, axis=1))
    per = 1.0 - sel / ((nrm + 1e-12) * (1.0 + 1e-12))
    m = mk_ref[...][:, 0]
    psum = jnp.sum(jnp.where(m > 0, per, 0.0))
    pcnt = jnp.sum(m.astype(jnp.float32))

    @pl.when(i == 0)
    def _():
        acc_ref[0] = 0.0
        acc_ref[1] = 0.0

    acc_ref[0] += psum
    acc_ref[1] += pcnt

    @pl.when(i == NB - 1)
    def _():
        out_ref[...] = jnp.full((1, 1), acc_ref[0] / acc_ref[1], jnp.float32)


def _embed_body(p_hbm, pidx_hbm, out_hbm, ibuf, rows, sem):
    c = lax.axis_index("c")
    s = lax.axis_index("s")
    wid = s * 2 + c

    def body(j, carry):
        pltpu.sync_copy(pidx_hbm.at[wid, j], ibuf)
        pltpu.async_copy(p_hbm.at[ibuf], rows, sem).wait()
        pltpu.sync_copy(rows, out_hbm.at[pl.ds(wid * 3 * CH + j * CH, CH)])
        return carry

    lax.fori_loop(0, 3, body, 0)


def _agg_body(hcat_hbm, gidx_hbm, dst_hbm, zeros_hbm, out_hbm,
              ibuf0, dbuf0, rows0, sem0, ibuf1, dbuf1, rows1, sem1, agg):
    c = lax.axis_index("c")
    s = lax.axis_index("s")
    wid = s * 2 + c

    pltpu.sync_copy(zeros_hbm, agg.at[pl.ds(s * ZR, ZR)])
    plsc.subcore_barrier()

    ibufs = (ibuf0, ibuf1)
    dbufs = (dbuf0, dbuf1)
    rowss = (rows0, rows1)
    sems = (sem0, sem1)

    for b in range(2):
        pltpu.sync_copy(gidx_hbm.at[wid, b], ibufs[b])
        pltpu.sync_copy(dst_hbm.at[wid, b], dbufs[b])
        pltpu.async_copy(hcat_hbm.at[ibufs[b]], rowss[b], sems[b])

    def body(i, carry):
        j0 = 2 * i
        for b in range(2):
            pltpu.make_async_copy(hcat_hbm.at[ibufs[b]], rowss[b],
                                  sems[b]).wait()
            pltpu.sync_copy(rowss[b], agg.at[dbufs[b]], add=True)
            pltpu.sync_copy(gidx_hbm.at[wid, j0 + 2 + b], ibufs[b])
            pltpu.sync_copy(dst_hbm.at[wid, j0 + 2 + b], dbufs[b])
            pltpu.async_copy(hcat_hbm.at[ibufs[b]], rowss[b], sems[b])
        return carry

    lax.fori_loop(0, (NCHUNK - 2) // 2, body, 0)

    for b in range(2):
        pltpu.make_async_copy(hcat_hbm.at[ibufs[b]], rowss[b], sems[b]).wait()
        pltpu.sync_copy(rowss[b], agg.at[dbufs[b]], add=True)

    plsc.subcore_barrier()

    pltpu.sync_copy(agg.at[pl.ds(s * ZR, ZR)],
                    out_hbm.at[c, pl.ds(s * ZR, ZR)])


def _embed_call(pair, pidx3):
    fn = pl.kernel(
        _embed_body,
        mesh=plsc.VectorSubcoreMesh(core_axis_name="c", subcore_axis_name="s"),
        out_type=jax.ShapeDtypeStruct((NP_EMB, D), jnp.float32),
        scratch_types=[
            pltpu.VMEM((CH,), jnp.int32),
            pltpu.VMEM((CH, D), jnp.float32),
            pltpu.SemaphoreType.DMA,
        ],
    )
    return fn(pair, pidx3)


def _agg_call(hcat, gidx3, dst3, zeros_hbm):
    fn = pl.kernel(
        _agg_body,
        mesh=plsc.VectorSubcoreMesh(core_axis_name="c", subcore_axis_name="s"),
        out_type=jax.ShapeDtypeStruct((2, AGG_ROWS, D), jnp.float32),
        scratch_types=[
            pltpu.VMEM((CH,), jnp.int32),
            pltpu.VMEM((CH,), jnp.int32),
            pltpu.VMEM((CH, D), jnp.float32),
            pltpu.SemaphoreType.DMA,
            pltpu.VMEM((CH,), jnp.int32),
            pltpu.VMEM((CH,), jnp.int32),
            pltpu.VMEM((CH, D), jnp.float32),
            pltpu.SemaphoreType.DMA,
            pltpu.VMEM_SHARED((AGG_ROWS, D), jnp.float32),
        ],
    )
    return fn(hcat, gidx3, dst3, zeros_hbm)


def _prep_call(a2, b2, src2, xm0f, xm1f):
    return pl.pallas_call(
        _prep_body,
        out_shape=(
            jax.ShapeDtypeStruct((EPAD // 128, 128), jnp.int32),
            jax.ShapeDtypeStruct((NP_EMB // 128, 128), jnp.int32),
        ),
    )(a2, b2, src2, xm0f, xm1f)


def _pair_call(e0, e1):
    return pl.pallas_call(
        _pair_body,
        out_shape=jax.ShapeDtypeStruct((NCLS * NCLS, D), jnp.float32),
    )(e0, e1)


def _hcat_call(h, e0, e1):
    return pl.pallas_call(
        _hcat_body,
        grid=(NB, 16),
        in_specs=[
            pl.BlockSpec((BN, D), lambda i, c: (i, 0)),
            pl.BlockSpec((8, D), lambda i, c: (0, 0)),
            pl.BlockSpec((8, D), lambda i, c: (0, 0)),
        ],
        out_specs=pl.BlockSpec((BN, D), lambda i, c: (c * NB + i, 0)),
        out_shape=jax.ShapeDtypeStruct((16 * N, D), jnp.float32),
    )(h, e0, e1)


def _mlp_call(pa, pb, h, w1, b1, w2, b2, relu_out):
    return pl.pallas_call(
        functools.partial(_mlp_body, relu_out=relu_out),
        grid=(NB,),
        in_specs=[
            pl.BlockSpec((BN, D), lambda i: (i, 0)),
            pl.BlockSpec((BN, D), lambda i: (i, 0)),
            pl.BlockSpec((BN, D), lambda i: (i, 0)),
            pl.BlockSpec((D, 2 * D), lambda i: (0, 0)),
            pl.BlockSpec((1, 2 * D), lambda i: (0, 0)),
            pl.BlockSpec((2 * D, D), lambda i: (0, 0)),
            pl.BlockSpec((1, D), lambda i: (0, 0)),
        ],
        out_specs=pl.BlockSpec((BN, D), lambda i: (i, 0)),
        out_shape=jax.ShapeDtypeStruct((N, D), jnp.float32),
    )(pa, pb, h, w1, b1, w2, b2)


def _mlp_mask_call(pa, pb, h, w1, b1, w2, b2, mk, me):
    return pl.pallas_call(
        _mlp_mask_body,
        grid=(NB,),
        in_specs=[
            pl.BlockSpec((BN, D), lambda i: (i, 0)),
            pl.BlockSpec((BN, D), lambda i: (i, 0)),
            pl.BlockSpec((BN, D), lambda i: (i, 0)),
            pl.BlockSpec((D, 2 * D), lambda i: (0, 0)),
            pl.BlockSpec((1, 2 * D), lambda i: (0, 0)),
            pl.BlockSpec((2 * D, D), lambda i: (0, 0)),
            pl.BlockSpec((1, D), lambda i: (0, 0)),
            pl.BlockSpec((BN, 1), lambda i: (i, 0)),
            pl.BlockSpec((1, D), lambda i: (0, 0)),
        ],
        out_specs=pl.BlockSpec((BN, D), lambda i: (i, 0)),
        out_shape=jax.ShapeDtypeStruct((N, D), jnp.float32),
    )(pa, pb, h, w1, b1, w2, b2, mk, me)


def _loss_call(hd, wo, bo, x0, mk):
    return pl.pallas_call(
        _loss_body,
        grid=(NB,),
        in_specs=[
            pl.BlockSpec((BN, D), lambda i: (i, 0)),
            pl.BlockSpec((D, NCLS), lambda i: (0, 0)),
            pl.BlockSpec((1, NCLS), lambda i: (0, 0)),
            pl.BlockSpec((BN, 1), lambda i: (i, 0)),
            pl.BlockSpec((BN, 1), lambda i: (i, 0)),
        ],
        out_specs=pl.BlockSpec((1, 1), lambda i: (0, 0)),
        out_shape=jax.ShapeDtypeStruct((1, 1), jnp.float32),
        scratch_shapes=[pltpu.SMEM((2,), jnp.float32)],
    )(hd, wo, bo, x0, mk)


def kernel(x, x_masked, edge_index, edge_attr, mask_tokens, batch,
           atom_emb0, atom_emb1, edge_emb0, edge_emb1,
           enc_W1, enc_b1, enc_W2, enc_b2,
           mask_emb, dec_W1, dec_b1, dec_W2, dec_b2, W_out, b_out):
    i32 = jnp.int32
    src = edge_index[0].astype(i32)
    dst = edge_index[1].astype(i32)
    ea = edge_attr[:, 0].astype(i32)
    eb = edge_attr[:, 1].astype(i32)

    epad = EPAD - E
    a2 = jnp.concatenate([ea, jnp.zeros((epad,), i32)]).reshape(EPAD // 128, 128)
    b2 = jnp.concatenate([eb, jnp.zeros((epad,), i32)]).reshape(EPAD // 128, 128)
    src2 = jnp.concatenate([src, jnp.zeros((epad,), i32)]).reshape(EPAD // 128, 128)
    pad_dst = N + jnp.arange(epad, dtype=i32) % (AGG_ROWS - N)
    dst3 = jnp.concatenate([dst, pad_dst]).reshape(NTILES, NCHUNK, CH)

    npad = NP_EMB - N
    xm0f = jnp.concatenate([x_masked[:, 0].astype(i32),
                            jnp.zeros((npad,), i32)]).reshape(NP_EMB // 128, 128)
    xm1f = jnp.concatenate([x_masked[:, 1].astype(i32),
                            jnp.zeros((npad,), i32)]).reshape(NP_EMB // 128, 128)

    gidx2, pidx2 = _prep_call(a2, b2, src2, xm0f, xm1f)
    gidx3 = gidx2.reshape(NTILES, NCHUNK, CH)
    pidx3 = pidx2.reshape(NTILES, 3, CH)

    pair = _pair_call(atom_emb0, atom_emb1)
    h = _embed_call(pair, pidx3)[:N]

    zeros_hbm = jnp.zeros((ZR, D), jnp.float32)
    mk = mask_tokens.astype(i32).reshape(N, 1)
    me = mask_emb.reshape(1, D)

    for l in range(L):
        hcat = _hcat_call(h, edge_emb0, edge_emb1)
        part = _agg_call(hcat, gidx3, dst3, zeros_hbm)
        w1 = enc_W1[l]
        b1 = enc_b1[l].reshape(1, 2 * D)
        w2 = enc_W2[l]
        b2_ = enc_b2[l].reshape(1, D)
        pa, pb = part[0, :N], part[1, :N]
        if l < L - 1:
            h = _mlp_call(pa, pb, h, w1, b1, w2, b2_, relu_out=True)
        else:
            h = _mlp_mask_call(pa, pb, h, w1, b1, w2, b2_, mk, me)

    hcat = _hcat_call(h, edge_emb0, edge_emb1)
    part = _agg_call(hcat, gidx3, dst3, zeros_hbm)
    hd = _mlp_call(part[0, :N], part[1, :N], h, dec_W1, dec_b1.reshape(1, 2 * D),
                   dec_W2, dec_b2.reshape(1, D), relu_out=False)

    loss = _loss_call(hd, W_out, b_out.reshape(1, NCLS),
                      x[:, 0].astype(i32).reshape(N, 1), mk)
    return loss[0, 0]


# speedup vs baseline: 1.3724x; 1.3724x over previous
import functools

import jax
import jax.numpy as jnp
from jax import lax
from jax.experimental import pallas as pl
from jax.experimental.pallas import tpu as pltpu
from jax.experimental.pallas import tpu_sc as plsc

N = 10000
E = 320000
D = 128
L = 5
NCLS = 120

NTILES = 32
CH = 128
NCHUNK = 79
EPT = NCHUNK * CH
EPAD = NTILES * EPT
NB = 10
BN = N // NB
AGG_ROWS = 10112
ZR = AGG_ROWS // 16
NP_EMB = NTILES * 3 * CH


def _prep_body(a_ref, b_ref, src_ref, xm0_ref, xm1_ref, gidx_ref, pidx_ref):
    gidx_ref[...] = (a_ref[...] * 4 + b_ref[...]) * N + src_ref[...]
    pidx_ref[...] = xm0_ref[...] * NCLS + xm1_ref[...]


def _pair_body(e0_ref, e1_ref, out_ref):
    t = e0_ref[...][:, None, :] + e1_ref[...][None, :, :]
    out_ref[...] = t.reshape(NCLS * NCLS, D)


def _hcat_body(h_ref, e0_ref, e1_ref, out_ref):
    c = pl.program_id(1)
    t = e0_ref[pl.ds(c // 4, 1), :] + e1_ref[pl.ds(c % 4, 1), :]
    out_ref[...] = jnp.maximum(h_ref[...] + t, 0.0)


def _mlp_body(pa_ref, pb_ref, h_ref, w1_ref, b1_ref, w2_ref, b2_ref, out_ref,
              *, relu_out):
    u = pa_ref[...] + pb_ref[...] + h_ref[...]
    z = jnp.maximum(
        jnp.dot(u, w1_ref[...], preferred_element_type=jnp.float32)
        + b1_ref[...], 0.0)
    o = jnp.dot(z, w2_ref[...], preferred_element_type=jnp.float32) + b2_ref[...]
    if relu_out:
        o = jnp.maximum(o, 0.0)
    out_ref[...] = o


def _mlp_mask_body(pa_ref, pb_ref, h_ref, w1_ref, b1_ref, w2_ref, b2_ref,
                   mk_ref, me_ref, out_ref):
    u = pa_ref[...] + pb_ref[...] + h_ref[...]
    z = jnp.maximum(
        jnp.dot(u, w1_ref[...], preferred_element_type=jnp.float32)
        + b1_ref[...], 0.0)
    o = jnp.dot(z, w2_ref[...], preferred_element_type=jnp.float32) + b2_ref[...]
    out_ref[...] = jnp.where(mk_ref[...] > 0, me_ref[...], o)


def _loss_body(hd_ref, wo_ref, bo_ref, x0_ref, mk_ref, out_ref, acc_ref):
    i = pl.program_id(0)
    logits = jnp.dot(hd_ref[...], wo_ref[...],
                     preferred_element_type=jnp.float32) + bo_ref[...]
    iota = lax.broadcasted_iota(jnp.int32, (BN, NCLS), 1)
    sel = jnp.sum(jnp.where(iota == x0_ref[...], logits, 0.0), axis=1)
    nrm = jnp.sqrt(jnp.sum(logits * logits, axis=1))
    per = 1.0 - sel / ((nrm + 1e-12) * (1.0 + 1e-12))
    m = mk_ref[...][:, 0]
    psum = jnp.sum(jnp.where(m > 0, per, 0.0))
    pcnt = jnp.sum(m.astype(jnp.float32))

    @pl.when(i == 0)
    def _():
        acc_ref[0] = 0.0
        acc_ref[1] = 0.0

    acc_ref[0] += psum
    acc_ref[1] += pcnt

    @pl.when(i == NB - 1)
    def _():
        out_ref[...] = jnp.full((1, 1), acc_ref[0] / acc_ref[1], jnp.float32)


def _embed_body(p_hbm, pidx_hbm, out_hbm, ibuf, rows, sem):
    c = lax.axis_index("c")
    s = lax.axis_index("s")
    wid = s * 2 + c

    def body(j, carry):
        pltpu.sync_copy(pidx_hbm.at[wid, j], ibuf)
        pltpu.async_copy(p_hbm.at[ibuf], rows, sem).wait()
        pltpu.sync_copy(rows, out_hbm.at[pl.ds(wid * 3 * CH + j * CH, CH)])
        return carry

    lax.fori_loop(0, 3, body, 0)


def _agg_body(hcat_hbm, gidx_hbm, dst_hbm, zeros_hbm, out_hbm,
              iall, dall, rows, sem, agg):
    c = lax.axis_index("c")
    s = lax.axis_index("s")
    wid = s * 2 + c

    pltpu.sync_copy(zeros_hbm, agg.at[pl.ds(s * ZR, ZR)])
    pltpu.sync_copy(gidx_hbm.at[wid], iall)
    pltpu.sync_copy(dst_hbm.at[wid], dall)
    plsc.subcore_barrier()

    def body(j, carry):
        pltpu.async_copy(hcat_hbm.at[iall.at[j]], rows, sem).wait()
        pltpu.sync_copy(rows, agg.at[dall.at[j]], add=True)
        return carry

    lax.fori_loop(0, NCHUNK, body, 0)
    plsc.subcore_barrier()

    pltpu.sync_copy(agg.at[pl.ds(s * ZR, ZR)],
                    out_hbm.at[c, pl.ds(s * ZR, ZR)])


def _embed_call(pair, pidx3):
    fn = pl.kernel(
        _embed_body,
        mesh=plsc.VectorSubcoreMesh(core_axis_name="c", subcore_axis_name="s"),
        out_type=jax.ShapeDtypeStruct((NP_EMB, D), jnp.float32),
        scratch_types=[
            pltpu.VMEM((CH,), jnp.int32),
            pltpu.VMEM((CH, D), jnp.float32),
            pltpu.SemaphoreType.DMA,
        ],
    )
    return fn(pair, pidx3)


def _agg_call(hcat, gidx3, dst3, zeros_hbm):
    fn = pl.kernel(
        _agg_body,
        mesh=plsc.VectorSubcoreMesh(core_axis_name="c", subcore_axis_name="s"),
        out_type=jax.ShapeDtypeStruct((2, AGG_ROWS, D), jnp.float32),
        scratch_types=[
            pltpu.VMEM((NCHUNK, CH), jnp.int32),
            pltpu.VMEM((NCHUNK, CH), jnp.int32),
            pltpu.VMEM((CH, D), jnp.float32),
            pltpu.SemaphoreType.DMA,
            pltpu.VMEM_SHARED((AGG_ROWS, D), jnp.float32),
        ],
    )
    return fn(hcat, gidx3, dst3, zeros_hbm)


def _prep_call(a2, b2, src2, xm0f, xm1f):
    return pl.pallas_call(
        _prep_body,
        out_shape=(
            jax.ShapeDtypeStruct((EPAD // 128, 128), jnp.int32),
            jax.ShapeDtypeStruct((NP_EMB // 128, 128), jnp.int32),
        ),
    )(a2, b2, src2, xm0f, xm1f)


def _pair_call(e0, e1):
    return pl.pallas_call(
        _pair_body,
        out_shape=jax.ShapeDtypeStruct((NCLS * NCLS, D), jnp.float32),
    )(e0, e1)


def _hcat_call(h, e0, e1):
    return pl.pallas_call(
        _hcat_body,
        grid=(NB, 16),
        in_specs=[
            pl.BlockSpec((BN, D), lambda i, c: (i, 0)),
            pl.BlockSpec((8, D), lambda i, c: (0, 0)),
            pl.BlockSpec((8, D), lambda i, c: (0, 0)),
        ],
        out_specs=pl.BlockSpec((BN, D), lambda i, c: (c * NB + i, 0)),
        out_shape=jax.ShapeDtypeStruct((16 * N, D), jnp.float32),
    )(h, e0, e1)


def _mlp_call(pa, pb, h, w1, b1, w2, b2, relu_out):
    return pl.pallas_call(
        functools.partial(_mlp_body, relu_out=relu_out),
        grid=(NB,),
        in_specs=[
            pl.BlockSpec((BN, D), lambda i: (i, 0)),
            pl.BlockSpec((BN, D), lambda i: (i, 0)),
            pl.BlockSpec((BN, D), lambda i: (i, 0)),
            pl.BlockSpec((D, 2 * D), lambda i: (0, 0)),
            pl.BlockSpec((1, 2 * D), lambda i: (0, 0)),
            pl.BlockSpec((2 * D, D), lambda i: (0, 0)),
            pl.BlockSpec((1, D), lambda i: (0, 0)),
        ],
        out_specs=pl.BlockSpec((BN, D), lambda i: (i, 0)),
        out_shape=jax.ShapeDtypeStruct((N, D), jnp.float32),
    )(pa, pb, h, w1, b1, w2, b2)


def _mlp_mask_call(pa, pb, h, w1, b1, w2, b2, mk, me):
    return pl.pallas_call(
        _mlp_mask_body,
        grid=(NB,),
        in_specs=[
            pl.BlockSpec((BN, D), lambda i: (i, 0)),
            pl.BlockSpec((BN, D), lambda i: (i, 0)),
            pl.BlockSpec((BN, D), lambda i: (i, 0)),
            pl.BlockSpec((D, 2 * D), lambda i: (0, 0)),
            pl.BlockSpec((1, 2 * D), lambda i: (0, 0)),
            pl.BlockSpec((2 * D, D), lambda i: (0, 0)),
            pl.BlockSpec((1, D), lambda i: (0, 0)),
            pl.BlockSpec((BN, 1), lambda i: (i, 0)),
            pl.BlockSpec((1, D), lambda i: (0, 0)),
        ],
        out_specs=pl.BlockSpec((BN, D), lambda i: (i, 0)),
        out_shape=jax.ShapeDtypeStruct((N, D), jnp.float32),
    )(pa, pb, h, w1, b1, w2, b2, mk, me)


def _loss_call(hd, wo, bo, x0, mk):
    return pl.pallas_call(
        _loss_body,
        grid=(NB,),
        in_specs=[
            pl.BlockSpec((BN, D), lambda i: (i, 0)),
            pl.BlockSpec((D, NCLS), lambda i: (0, 0)),
            pl.BlockSpec((1, NCLS), lambda i: (0, 0)),
            pl.BlockSpec((BN, 1), lambda i: (i, 0)),
            pl.BlockSpec((BN, 1), lambda i: (i, 0)),
        ],
        out_specs=pl.BlockSpec((1, 1), lambda i: (0, 0)),
        out_shape=jax.ShapeDtypeStruct((1, 1), jnp.float32),
        scratch_shapes=[pltpu.SMEM((2,), jnp.float32)],
    )(hd, wo, bo, x0, mk)


def kernel(x, x_masked, edge_index, edge_attr, mask_tokens, batch,
           atom_emb0, atom_emb1, edge_emb0, edge_emb1,
           enc_W1, enc_b1, enc_W2, enc_b2,
           mask_emb, dec_W1, dec_b1, dec_W2, dec_b2, W_out, b_out):
    i32 = jnp.int32
    src = edge_index[0].astype(i32)
    dst = edge_index[1].astype(i32)
    ea = edge_attr[:, 0].astype(i32)
    eb = edge_attr[:, 1].astype(i32)

    epad = EPAD - E
    a2 = jnp.concatenate([ea, jnp.zeros((epad,), i32)]).reshape(EPAD // 128, 128)
    b2 = jnp.concatenate([eb, jnp.zeros((epad,), i32)]).reshape(EPAD // 128, 128)
    src2 = jnp.concatenate([src, jnp.zeros((epad,), i32)]).reshape(EPAD // 128, 128)
    pad_dst = N + jnp.arange(epad, dtype=i32) % (AGG_ROWS - N)
    dst3 = jnp.concatenate([dst, pad_dst]).reshape(NTILES, NCHUNK, CH)

    npad = NP_EMB - N
    xm0f = jnp.concatenate([x_masked[:, 0].astype(i32),
                            jnp.zeros((npad,), i32)]).reshape(NP_EMB // 128, 128)
    xm1f = jnp.concatenate([x_masked[:, 1].astype(i32),
                            jnp.zeros((npad,), i32)]).reshape(NP_EMB // 128, 128)

    gidx2, pidx2 = _prep_call(a2, b2, src2, xm0f, xm1f)
    gidx3 = gidx2.reshape(NTILES, NCHUNK, CH)
    pidx3 = pidx2.reshape(NTILES, 3, CH)

    pair = _pair_call(atom_emb0, atom_emb1)
    h = _embed_call(pair, pidx3)[:N]

    zeros_hbm = jnp.zeros((ZR, D), jnp.float32)
    mk = mask_tokens.astype(i32).reshape(N, 1)
    me = mask_emb.reshape(1, D)

    for l in range(L):
        hcat = _hcat_call(h, edge_emb0, edge_emb1)
        part = _agg_call(hcat, gidx3, dst3, zeros_hbm)
        w1 = enc_W1[l]
        b1 = enc_b1[l].reshape(1, 2 * D)
        w2 = enc_W2[l]
        b2_ = enc_b2[l].reshape(1, D)
        pa, pb = part[0, :N], part[1, :N]
        if l < L - 1:
            h = _mlp_call(pa, pb, h, w1, b1, w2, b2_, relu_out=True)
        else:
            h = _mlp_mask_call(pa, pb, h, w1, b1, w2, b2_, mk, me)

    hcat = _hcat_call(h, edge_emb0, edge_emb1)
    part = _agg_call(hcat, gidx3, dst3, zeros_hbm)
    hd = _mlp_call(part[0, :N], part[1, :N], h, dec_W1, dec_b1.reshape(1, 2 * D),
                   dec_W2, dec_b2.reshape(1, D), relu_out=False)

    loss = _loss_call(hd, W_out, b_out.reshape(1, NCLS),
                      x[:, 0].astype(i32).reshape(N, 1), mk)
    return loss[0, 0]


# speedup vs baseline: 1.3724x; 1.0000x over previous
import functools

import jax
import jax.numpy as jnp
from jax import lax
from jax.experimental import pallas as pl
from jax.experimental.pallas import tpu as pltpu
from jax.experimental.pallas import tpu_sc as plsc

N = 10000
E = 320000
D = 128
L = 5
NCLS = 120

NTILES = 32
CH = 128
NCHUNK = 79
EPT = NCHUNK * CH
EPAD = NTILES * EPT
NB = 10
BN = N // NB
AGG_ROWS = 10112
ZR = AGG_ROWS // 16
NP_EMB = NTILES * 3 * CH


def _prep_body(a_ref, b_ref, src_ref, xm0_ref, xm1_ref, gidx_ref, pidx_ref):
    gidx_ref[...] = (a_ref[...] * 4 + b_ref[...]) * N + src_ref[...]
    pidx_ref[...] = xm0_ref[...] * NCLS + xm1_ref[...]


def _pair_body(e0_ref, e1_ref, out_ref):
    t = e0_ref[...][:, None, :] + e1_ref[...][None, :, :]
    out_ref[...] = t.reshape(NCLS * NCLS, D)


def _hcat_body(h_ref, e0_ref, e1_ref, out_ref):
    c = pl.program_id(1)
    t = e0_ref[pl.ds(c // 4, 1), :] + e1_ref[pl.ds(c % 4, 1), :]
    out_ref[...] = jnp.maximum(h_ref[...] + t, 0.0)


def _mlp_body(pa_ref, pb_ref, h_ref, w1_ref, b1_ref, w2_ref, b2_ref, out_ref,
              *, relu_out):
    u = pa_ref[...] + pb_ref[...] + h_ref[...]
    z = jnp.maximum(
        jnp.dot(u, w1_ref[...], preferred_element_type=jnp.float32)
        + b1_ref[...], 0.0)
    o = jnp.dot(z, w2_ref[...], preferred_element_type=jnp.float32) + b2_ref[...]
    if relu_out:
        o = jnp.maximum(o, 0.0)
    out_ref[...] = o


def _mlp_mask_body(pa_ref, pb_ref, h_ref, w1_ref, b1_ref, w2_ref, b2_ref,
                   mk_ref, me_ref, out_ref):
    u = pa_ref[...] + pb_ref[...] + h_ref[...]
    z = jnp.maximum(
        jnp.dot(u, w1_ref[...], preferred_element_type=jnp.float32)
        + b1_ref[...], 0.0)
    o = jnp.dot(z, w2_ref[...], preferred_element_type=jnp.float32) + b2_ref[...]
    out_ref[...] = jnp.where(mk_ref[...] > 0, me_ref[...], o)


def _loss_body(hd_ref, wo_ref, bo_ref, x0_ref, mk_ref, out_ref, acc_ref):
    i = pl.program_id(0)
    logits = jnp.dot(hd_ref[...], wo_ref[...],
                     preferred_element_type=jnp.float32) + bo_ref[...]
    iota = lax.broadcasted_iota(jnp.int32, (BN, NCLS), 1)
    sel = jnp.sum(jnp.where(iota == x0_ref[...], logits, 0.0), axis=1)
    nrm = jnp.sqrt(jnp.sum(logits * logits, axis=1))
    per = 1.0 - sel / ((nrm + 1e-12) * (1.0 + 1e-12))
    m = mk_ref[...][:, 0]
    psum = jnp.sum(jnp.where(m > 0, per, 0.0))
    pcnt = jnp.sum(m.astype(jnp.float32))

    @pl.when(i == 0)
    def _():
        acc_ref[0] = 0.0
        acc_ref[1] = 0.0

    acc_ref[0] += psum
    acc_ref[1] += pcnt

    @pl.when(i == NB - 1)
    def _():
        out_ref[...] = jnp.full((1, 1), acc_ref[0] / acc_ref[1], jnp.float32)


def _embed_body(p_hbm, pidx_hbm, out_hbm, iall, rows, sem):
    c = lax.axis_index("c")
    s = lax.axis_index("s")
    wid = s * 2 + c

    pltpu.sync_copy(pidx_hbm.at[wid], iall)

    def body(j, carry):
        pltpu.async_copy(p_hbm.at[iall.at[j]], rows, sem).wait()
        pltpu.sync_copy(rows, out_hbm.at[pl.ds(wid * 3 * CH + j * CH, CH)])
        return carry

    lax.fori_loop(0, 3, body, 0)


def _agg_body(hcat_hbm, gidx_hbm, dst_hbm, zeros_hbm, out_hbm,
              iall, dall, rows, sem, agg):
    c = lax.axis_index("c")
    s = lax.axis_index("s")
    wid = s * 2 + c

    pltpu.sync_copy(zeros_hbm, agg.at[pl.ds(s * ZR, ZR)])
    pltpu.sync_copy(gidx_hbm.at[wid], iall)
    pltpu.sync_copy(dst_hbm.at[wid], dall)
    plsc.subcore_barrier()

    def body(j, carry):
        pltpu.async_copy(hcat_hbm.at[iall.at[j]], rows, sem).wait()
        pltpu.sync_copy(rows, agg.at[dall.at[j]], add=True)
        return carry

    lax.fori_loop(0, NCHUNK, body, 0)
    plsc.subcore_barrier()

    pltpu.sync_copy(agg.at[pl.ds(s * ZR, ZR)],
                    out_hbm.at[c, pl.ds(s * ZR, ZR)])


def _embed_call(pair, pidx3):
    fn = pl.kernel(
        _embed_body,
        mesh=plsc.VectorSubcoreMesh(core_axis_name="c", subcore_axis_name="s"),
        out_type=jax.ShapeDtypeStruct((NP_EMB, D), jnp.float32),
        scratch_types=[
            pltpu.VMEM((3, CH), jnp.int32),
            pltpu.VMEM((CH, D), jnp.float32),
            pltpu.SemaphoreType.DMA,
        ],
    )
    return fn(pair, pidx3)


def _agg_call(hcat, gidx3, dst3, zeros_hbm):
    fn = pl.kernel(
        _agg_body,
        mesh=plsc.VectorSubcoreMesh(core_axis_name="c", subcore_axis_name="s"),
        out_type=jax.ShapeDtypeStruct((2, AGG_ROWS, D), jnp.float32),
        scratch_types=[
            pltpu.VMEM((NCHUNK, CH), jnp.int32),
            pltpu.VMEM((NCHUNK, CH), jnp.int32),
            pltpu.VMEM((CH, D), jnp.float32),
            pltpu.SemaphoreType.DMA,
            pltpu.VMEM_SHARED((AGG_ROWS, D), jnp.float32),
        ],
    )
    return fn(hcat, gidx3, dst3, zeros_hbm)


def _prep_call(a2, b2, src2, xm0f, xm1f):
    return pl.pallas_call(
        _prep_body,
        out_shape=(
            jax.ShapeDtypeStruct((EPAD // 128, 128), jnp.int32),
            jax.ShapeDtypeStruct((NP_EMB // 128, 128), jnp.int32),
        ),
    )(a2, b2, src2, xm0f, xm1f)


def _pair_call(e0, e1):
    return pl.pallas_call(
        _pair_body,
        out_shape=jax.ShapeDtypeStruct((NCLS * NCLS, D), jnp.float32),
    )(e0, e1)


def _hcat_call(h, e0, e1):
    return pl.pallas_call(
        _hcat_body,
        grid=(NB, 16),
        in_specs=[
            pl.BlockSpec((BN, D), lambda i, c: (i, 0)),
            pl.BlockSpec((8, D), lambda i, c: (0, 0)),
            pl.BlockSpec((8, D), lambda i, c: (0, 0)),
        ],
        out_specs=pl.BlockSpec((BN, D), lambda i, c: (c * NB + i, 0)),
        out_shape=jax.ShapeDtypeStruct((16 * N, D), jnp.float32),
    )(h, e0, e1)


def _mlp_call(pa, pb, h, w1, b1, w2, b2, relu_out):
    return pl.pallas_call(
        functools.partial(_mlp_body, relu_out=relu_out),
        grid=(NB,),
        in_specs=[
            pl.BlockSpec((BN, D), lambda i: (i, 0)),
            pl.BlockSpec((BN, D), lambda i: (i, 0)),
            pl.BlockSpec((BN, D), lambda i: (i, 0)),
            pl.BlockSpec((D, 2 * D), lambda i: (0, 0)),
            pl.BlockSpec((1, 2 * D), lambda i: (0, 0)),
            pl.BlockSpec((2 * D, D), lambda i: (0, 0)),
            pl.BlockSpec((1, D), lambda i: (0, 0)),
        ],
        out_specs=pl.BlockSpec((BN, D), lambda i: (i, 0)),
        out_shape=jax.ShapeDtypeStruct((N, D), jnp.float32),
    )(pa, pb, h, w1, b1, w2, b2)


def _mlp_mask_call(pa, pb, h, w1, b1, w2, b2, mk, me):
    return pl.pallas_call(
        _mlp_mask_body,
        grid=(NB,),
        in_specs=[
            pl.BlockSpec((BN, D), lambda i: (i, 0)),
            pl.BlockSpec((BN, D), lambda i: (i, 0)),
            pl.BlockSpec((BN, D), lambda i: (i, 0)),
            pl.BlockSpec((D, 2 * D), lambda i: (0, 0)),
            pl.BlockSpec((1, 2 * D), lambda i: (0, 0)),
            pl.BlockSpec((2 * D, D), lambda i: (0, 0)),
            pl.BlockSpec((1, D), lambda i: (0, 0)),
            pl.BlockSpec((BN, 1), lambda i: (i, 0)),
            pl.BlockSpec((1, D), lambda i: (0, 0)),
        ],
        out_specs=pl.BlockSpec((BN, D), lambda i: (i, 0)),
        out_shape=jax.ShapeDtypeStruct((N, D), jnp.float32),
    )(pa, pb, h, w1, b1, w2, b2, mk, me)


def _loss_call(hd, wo, bo, x0, mk):
    return pl.pallas_call(
        _loss_body,
        grid=(NB,),
        in_specs=[
            pl.BlockSpec((BN, D), lambda i: (i, 0)),
            pl.BlockSpec((D, NCLS), lambda i: (0, 0)),
            pl.BlockSpec((1, NCLS), lambda i: (0, 0)),
            pl.BlockSpec((BN, 1), lambda i: (i, 0)),
            pl.BlockSpec((BN, 1), lambda i: (i, 0)),
        ],
        out_specs=pl.BlockSpec((1, 1), lambda i: (0, 0)),
        out_shape=jax.ShapeDtypeStruct((1, 1), jnp.float32),
        scratch_shapes=[pltpu.SMEM((2,), jnp.float32)],
    )(hd, wo, bo, x0, mk)


def kernel(x, x_masked, edge_index, edge_attr, mask_tokens, batch,
           atom_emb0, atom_emb1, edge_emb0, edge_emb1,
           enc_W1, enc_b1, enc_W2, enc_b2,
           mask_emb, dec_W1, dec_b1, dec_W2, dec_b2, W_out, b_out):
    i32 = jnp.int32
    src = edge_index[0].astype(i32)
    dst = edge_index[1].astype(i32)
    ea = edge_attr[:, 0].astype(i32)
    eb = edge_attr[:, 1].astype(i32)

    epad = EPAD - E
    a2 = jnp.concatenate([ea, jnp.zeros((epad,), i32)]).reshape(EPAD // 128, 128)
    b2 = jnp.concatenate([eb, jnp.zeros((epad,), i32)]).reshape(EPAD // 128, 128)
    src2 = jnp.concatenate([src, jnp.zeros((epad,), i32)]).reshape(EPAD // 128, 128)
    pad_dst = N + jnp.arange(epad, dtype=i32) % (AGG_ROWS - N)
    dst3 = jnp.concatenate([dst, pad_dst]).reshape(NTILES, NCHUNK, CH)

    npad = NP_EMB - N
    xm0f = jnp.concatenate([x_masked[:, 0].astype(i32),
                            jnp.zeros((npad,), i32)]).reshape(NP_EMB // 128, 128)
    xm1f = jnp.concatenate([x_masked[:, 1].astype(i32),
                            jnp.zeros((npad,), i32)]).reshape(NP_EMB // 128, 128)

    gidx2, pidx2 = _prep_call(a2, b2, src2, xm0f, xm1f)
    gidx3 = gidx2.reshape(NTILES, NCHUNK, CH)
    pidx3 = pidx2.reshape(NTILES, 3, CH)

    pair = _pair_call(atom_emb0, atom_emb1)
    h = _embed_call(pair, pidx3)[:N]

    zeros_hbm = jnp.zeros((ZR, D), jnp.float32)
    mk = mask_tokens.astype(i32).reshape(N, 1)
    me = mask_emb.reshape(1, D)

    for l in range(L):
        hcat = _hcat_call(h, edge_emb0, edge_emb1)
        part = _agg_call(hcat, gidx3, dst3, zeros_hbm)
        w1 = enc_W1[l]
        b1 = enc_b1[l].reshape(1, 2 * D)
        w2 = enc_W2[l]
        b2_ = enc_b2[l].reshape(1, D)
        pa, pb = part[0, :N], part[1, :N]
        if l < L - 1:
            h = _mlp_call(pa, pb, h, w1, b1, w2, b2_, relu_out=True)
        else:
            h = _mlp_mask_call(pa, pb, h, w1, b1, w2, b2_, mk, me)

    hcat = _hcat_call(h, edge_emb0, edge_emb1)
    part = _agg_call(hcat, gidx3, dst3, zeros_hbm)
    hd = _mlp_call(part[0, :N], part[1, :N], h, dec_W1, dec_b1.reshape(1, 2 * D),
                   dec_W2, dec_b2.reshape(1, D), relu_out=False)

    loss = _loss_call(hd, W_out, b_out.reshape(1, NCLS),
                      x[:, 0].astype(i32).reshape(N, 1), mk)
    return loss[0, 0]
